# baseline (speedup 1.0000x reference)
import functools

import jax
import jax.numpy as jnp
import numpy as np
from jax import lax
from jax.experimental import pallas as pl
from jax.experimental.pallas import tpu as pltpu
from jax.experimental.pallas import tpu_sc as plsc

VOCAB = 1000000
D = 64
BATCH = 4096
SEQ = 200
T = BATCH * SEQ

NC, NS = 2, 16
NW = NC * NS
CH = 2 * SEQ
G = 4
CG = CH // G
NCHUNK = T // CH
CPW = NCHUNK // NW


def _pe_table():
    pe = np.array(
        [[pos / np.power(10000, 2 * (j // 2) / D) for j in range(D)]
         if pos != 0 else np.zeros(D) for pos in range(SEQ)])
    pe[1:, 0::2] = np.sin(pe[1:, 0::2])
    pe[1:, 1::2] = np.cos(pe[1:, 1::2])
    return jnp.asarray(pe, dtype=jnp.float32)


_MESH = plsc.VectorSubcoreMesh(
    core_axis_name="c", subcore_axis_name="s", num_cores=NC, num_subcores=NS)


@functools.partial(
    pl.kernel,
    out_type=jax.ShapeDtypeStruct((T, D), jnp.float32),
    mesh=_MESH,
    scratch_types=[
        pltpu.VMEM((G, CG), jnp.int32),
        pltpu.VMEM((CH, D), jnp.float32),
        pltpu.VMEM((SEQ, D), jnp.float32),
        pltpu.SemaphoreType.DMA,
    ],
    compiler_params=pltpu.CompilerParams(use_tc_tiling_on_sc=False),
)
def _embed(x_hbm, table_hbm, pe_hbm, out_hbm, idx_v, rows_v, pe_v, sem):
    wid = lax.axis_index("s") * NC + lax.axis_index("c")
    pltpu.sync_copy(pe_hbm, pe_v)

    def chunk_body(c, _):
        g = wid * CPW + c
        pltpu.sync_copy(x_hbm.at[g], idx_v)
        for j in range(G):
            pltpu.async_copy(
                table_hbm.at[idx_v.at[j]],
                rows_v.at[pl.ds(j * CG, CG)], sem).wait()

        def tok_body(t, _):
            for k in range(D // 16):
                sl = pl.ds(k * 16, 16)
                p = pe_v[t, sl]
                rows_v[t, sl] += p
                rows_v[t + SEQ, sl] += p
            return ()

        lax.fori_loop(0, SEQ, tok_body, ())
        pltpu.sync_copy(rows_v, out_hbm.at[pl.ds(g * CH, CH)])
        return ()

    lax.fori_loop(0, CPW, chunk_body, ())


def kernel(x, table):
    out = _embed(x.reshape(NCHUNK, G, CG), table, _pe_table())
    return out.reshape(BATCH, SEQ, D)


# speedup vs baseline: 1.2074x; 1.2074x over previous
import functools

import jax
import jax.numpy as jnp
import numpy as np
from jax import lax
from jax.experimental import pallas as pl
from jax.experimental.pallas import tpu as pltpu
from jax.experimental.pallas import tpu_sc as plsc

VOCAB = 1000000
D = 64
BATCH = 4096
SEQ = 200
T = BATCH * SEQ

NC, NS = 2, 16
NW = NC * NS
CH = 2 * SEQ
G = 4
CG = CH // G
NCHUNK = T // CH
CPW = NCHUNK // NW


def _pe_table():
    pe = np.array(
        [[pos / np.power(10000, 2 * (j // 2) / D) for j in range(D)]
         if pos != 0 else np.zeros(D) for pos in range(SEQ)])
    pe[1:, 0::2] = np.sin(pe[1:, 0::2])
    pe[1:, 1::2] = np.cos(pe[1:, 1::2])
    return jnp.asarray(pe, dtype=jnp.float32)


_MESH = plsc.VectorSubcoreMesh(
    core_axis_name="c", subcore_axis_name="s", num_cores=NC, num_subcores=NS)


@functools.partial(
    pl.kernel,
    out_type=jax.ShapeDtypeStruct((T, D), jnp.float32),
    mesh=_MESH,
    scratch_types=[
        pltpu.VMEM((CPW * G, CG), jnp.int32),
        pltpu.VMEM((CH, D), jnp.float32),
        pltpu.VMEM((CH, D), jnp.float32),
        pltpu.VMEM((SEQ, D), jnp.float32),
        pltpu.SemaphoreType.DMA,
        pltpu.SemaphoreType.DMA,
        pltpu.SemaphoreType.DMA,
        pltpu.SemaphoreType.DMA,
    ],
    compiler_params=pltpu.CompilerParams(use_tc_tiling_on_sc=False),
)
def _embed(x_hbm, table_hbm, pe_hbm, out_hbm,
           idx_v, rows0, rows1, pe_v, sg0, sg1, so0, so1):
    wid = lax.axis_index("s") * NC + lax.axis_index("c")
    base = wid * CPW
    rows = [rows0, rows1]
    sg = [sg0, sg1]
    so = [so0, so1]

    pltpu.sync_copy(pe_hbm, pe_v)
    pltpu.sync_copy(x_hbm.at[pl.ds(base * G, CPW * G)], idx_v)

    def issue_gathers(c, b):
        for j in range(G):
            pltpu.async_copy(
                table_hbm.at[idx_v.at[c * G + j]],
                rows[b].at[pl.ds(j * CG, CG)], sg[b])

    issue_gathers(0, 0)

    def pair_body(cc, _):
        for b in range(2):
            c = 2 * cc + b
            pltpu.make_async_copy(
                out_hbm.at[pl.ds((base + c) * CH, CH)], rows[b], sg[b]).wait()

            @pl.when(c + 1 < CPW)
            def _():
                @pl.when(c >= 1)
                def _():
                    pltpu.make_async_copy(
                        rows[1 - b],
                        out_hbm.at[pl.ds((base + c - 1) * CH, CH)],
                        so[1 - b]).wait()
                issue_gathers(c + 1, 1 - b)

            def tok_body(t, _):
                for k in range(D // 16):
                    sl = pl.ds(k * 16, 16)
                    p = pe_v[t, sl]
                    rows[b][t, sl] += p
                    rows[b][t + SEQ, sl] += p
                return ()

            lax.fori_loop(0, SEQ, tok_body, ())
            pltpu.async_copy(
                rows[b], out_hbm.at[pl.ds((base + c) * CH, CH)], so[b])
        return ()

    lax.fori_loop(0, CPW // 2, pair_body, ())
    pltpu.make_async_copy(
        rows[0], out_hbm.at[pl.ds(base * CH, CH)], so[0]).wait()
    pltpu.make_async_copy(
        rows[1], out_hbm.at[pl.ds(base * CH, CH)], so[1]).wait()


def kernel(x, table):
    out = _embed(x.reshape(NCHUNK * G, CG), table, _pe_table())
    return out.reshape(BATCH, SEQ, D)
